# baseline (speedup 1.0000x reference)
import functools

import jax
import jax.numpy as jnp
from jax import lax
from jax.experimental import pallas as pl
from jax.experimental.pallas import tpu as pltpu
from jax.experimental.pallas import tpu_sc as plsc

_VOCAB = 10000
_N_TOK = 4096
_ROW_W = 32
_NW = 32
_L = 16
_TPW = _N_TOK // _NW
_G = _TPW // _L

_mesh = plsc.VectorSubcoreMesh(core_axis_name="c", subcore_axis_name="s")


@functools.partial(
    pl.kernel,
    out_type=jax.ShapeDtypeStruct((_NW * 2 * _L,), jnp.float32),
    mesh=_mesh,
    compiler_params=pltpu.CompilerParams(
        needs_layout_passes=False, use_tc_tiling_on_sc=True),
    scratch_types=[
        pltpu.VMEM((_ROW_W, _N_TOK // _ROW_W), jnp.int32),
        pltpu.VMEM((_TPW,), jnp.float32),
        pltpu.VMEM((_TPW,), jnp.int32),
        pltpu.VMEM((_TPW, 128), jnp.float32),
        pltpu.VMEM((2 * _L,), jnp.float32),
        pltpu.SemaphoreType.DMA,
        pltpu.SemaphoreType.DMA,
        pltpu.SemaphoreType.DMA,
    ],
)
def _lm_partials(txtT_hbm, vis_hbm, tgtT_hbm, out_hbm,
                 tgtT_v, vis_v, idx_v, gat_v, acc_v, sem_t, sem_v, sem_g):
    wid = lax.axis_index("s") * 2 + lax.axis_index("c")
    base = wid * _TPW
    cp_t = pltpu.async_copy(tgtT_hbm, tgtT_v, sem_t)
    cp_v = pltpu.async_copy(vis_hbm.at[pl.ds(base, _TPW)], vis_v, sem_v)
    cp_t.wait()

    lane = lax.iota(jnp.int32, _L)
    den_acc = jnp.zeros((_L,), jnp.float32)
    txt_masks = []
    vis_masks = []
    gathers = []
    for j in range(_G):
        tok = base + lane + (j * _L)
        cur = plsc.load_gather(tgtT_v, [tok % _ROW_W, tok // _ROW_W])
        pv = jnp.maximum(tok - 1, 0)
        prev = plsc.load_gather(tgtT_v, [pv % _ROW_W, pv // _ROW_W])
        vis_m = cur > _VOCAB
        first = (tok % _ROW_W) == 0
        txt_m = jnp.logical_and(jnp.logical_or(first, prev > 0),
                                jnp.logical_not(vis_m))
        tz = jnp.where(vis_m, 0, cur)
        idx_v[pl.ds(j * _L, _L)] = tz
        den_acc = (den_acc + vis_m.astype(jnp.float32)
                   + txt_m.astype(jnp.float32))
        txt_masks.append(txt_m)
        vis_masks.append(vis_m)

    gathers.append(pltpu.async_copy(
        txtT_hbm.at[idx_v, pl.ds(base, 128)], gat_v, sem_g))

    cp_v.wait()
    num_acc = jnp.zeros((_L,), jnp.float32)
    for j in range(_G):
        num_acc = num_acc + jnp.where(vis_masks[j],
                                      vis_v[pl.ds(j * _L, _L)], 0.0)

    gathers[0].wait()
    for j in range(_G):
        local = lane + (j * _L)
        vals = plsc.load_gather(gat_v, [local, local])
        num_acc = num_acc + jnp.where(txt_masks[j], vals, 0.0)

    acc_v[pl.ds(0, _L)] = num_acc
    acc_v[pl.ds(_L, _L)] = den_acc
    pltpu.sync_copy(acc_v, out_hbm.at[pl.ds(wid * 2 * _L, 2 * _L)])


def kernel(txt_input, vis_input, target):
    txtT = txt_input.T
    vis_flat = vis_input.reshape(-1)
    tgtT = target.astype(jnp.int32).T
    parts = _lm_partials(txtT, vis_flat, tgtT).reshape(_NW, 2, _L)
    num = jnp.sum(parts[:, 0, :])
    den = jnp.sum(parts[:, 1, :])
    return -(num / den)


# speedup vs baseline: 1.0540x; 1.0540x over previous
import functools

import jax
import jax.numpy as jnp
from jax import lax
from jax.experimental import pallas as pl
from jax.experimental.pallas import tpu as pltpu
from jax.experimental.pallas import tpu_sc as plsc

_VOCAB = 10000
_N_TOK = 4096
_ROW_W = 32
_NW = 32
_L = 16
_TPW = _N_TOK // _NW
_G = _TPW // _L

_mesh = plsc.VectorSubcoreMesh(core_axis_name="c", subcore_axis_name="s")


@functools.partial(
    pl.kernel,
    out_type=jax.ShapeDtypeStruct((2 * _NW * _L,), jnp.float32),
    mesh=_mesh,
    compiler_params=pltpu.CompilerParams(
        needs_layout_passes=False, use_tc_tiling_on_sc=True),
    scratch_types=[
        pltpu.VMEM((_TPW,), jnp.int32),
        pltpu.VMEM((_TPW,), jnp.float32),
        pltpu.VMEM((_TPW,), jnp.int32),
        pltpu.VMEM((_TPW, 128), jnp.float32),
        pltpu.VMEM((2 * _L,), jnp.float32),
        pltpu.SemaphoreType.DMA,
        pltpu.SemaphoreType.DMA,
        pltpu.SemaphoreType.DMA,
    ],
)
def _lm_partials(txtT_hbm, vis_hbm, tgt_hbm, out_hbm,
                 tgt_v, vis_v, idx_v, gat_v, acc_v, sem_t, sem_v, sem_g):
    wid = lax.axis_index("s") * 2 + lax.axis_index("c")
    base = wid * _TPW
    cp_t = pltpu.async_copy(tgt_hbm.at[pl.ds(base, _TPW)], tgt_v, sem_t)
    cp_v = pltpu.async_copy(vis_hbm.at[pl.ds(base, _TPW)], vis_v, sem_v)
    cp_t.wait()

    lane = lax.iota(jnp.int32, _L)
    num_acc = jnp.zeros((_L,), jnp.float32)
    den_acc = jnp.zeros((_L,), jnp.float32)
    txt_masks = []
    vis_masks = []
    for j in range(_G):
        local = lane + (j * _L)
        cur = tgt_v[pl.ds(j * _L, _L)]
        prev = plsc.load_gather(tgt_v, [jnp.maximum(local - 1, 0)])
        vis_m = cur > _VOCAB
        first = (local % _ROW_W) == 0
        txt_m = jnp.logical_and(jnp.logical_or(first, prev > 0),
                                jnp.logical_not(vis_m))
        idx_v[pl.ds(j * _L, _L)] = jnp.where(vis_m, 0, cur)
        den_acc = (den_acc + vis_m.astype(jnp.float32)
                   + txt_m.astype(jnp.float32))
        txt_masks.append(txt_m)
        vis_masks.append(vis_m)

    cp_g = pltpu.async_copy(
        txtT_hbm.at[idx_v, pl.ds(base, 128)], gat_v, sem_g)

    cp_v.wait()
    for j in range(_G):
        num_acc = num_acc + jnp.where(vis_masks[j],
                                      vis_v[pl.ds(j * _L, _L)], 0.0)

    cp_g.wait()
    for j in range(_G):
        local = lane + (j * _L)
        vals = plsc.load_gather(gat_v, [local, local])
        num_acc = num_acc + jnp.where(txt_masks[j], vals, 0.0)

    acc_v[pl.ds(0, _L)] = num_acc
    acc_v[pl.ds(_L, _L)] = den_acc
    pltpu.sync_copy(acc_v.at[pl.ds(0, _L)],
                    out_hbm.at[pl.ds(wid * _L, _L)])
    pltpu.sync_copy(acc_v.at[pl.ds(_L, _L)],
                    out_hbm.at[pl.ds(_NW * _L + wid * _L, _L)])


def kernel(txt_input, vis_input, target):
    txtT = txt_input.T
    vis_flat = vis_input.reshape(-1)
    tgt_flat = target.reshape(-1).astype(jnp.int32)
    parts = _lm_partials(txtT, vis_flat, tgt_flat)
    num = jnp.sum(parts[: _NW * _L])
    den = jnp.sum(parts[_NW * _L:])
    return -(num / den)
